# baseline (speedup 1.0000x reference)
import jax
import jax.numpy as jnp
from jax.experimental import pallas as pl
from jax.experimental.pallas import tpu as pltpu

_N_M = 800
_N_D = 400
_DIM = 32
_N_REL = 8
_N_BASES = 4


def _rgcn_kernel(msim_ref, dsim_ref, tern2d_ref,
                 xm_ref, xd_ref,
                 basis0_ref, slw0_ref, bias0_ref,
                 basis1_ref, slw1_ref, bias1_ref,
                 coeff0_ref, coeff1_ref,
                 outm_ref, outd_ref):
    f32 = jnp.float32

    a_m = (msim_ref[...] > 0.3).astype(f32)
    a_d = (dsim_ref[...] > 0.3).astype(f32)
    t2 = (tern2d_ref[...] > 0.5).astype(f32)

    x_m = xm_ref[...]
    x_d = xd_ref[...]

    def dot(a, b):
        return jax.lax.dot(a, b, preferred_element_type=f32)

    for basis_ref, slw_ref, bias_ref, coeff_ref in (
            (basis0_ref, slw0_ref, bias0_ref, coeff0_ref),
            (basis1_ref, slw1_ref, bias1_ref, coeff1_ref)):
        w = []
        for r in range(_N_REL):
            wr = coeff_ref[r, 0] * basis_ref[0]
            for b in range(1, _N_BASES):
                wr = wr + coeff_ref[r, b] * basis_ref[b]
            w.append(wr)

        slw = slw_ref[...]
        out_m = dot(x_m, slw)
        out_d = dot(x_d, slw)
        def dotT(a, b):
            return jax.lax.dot_general(
                a, b, dimension_numbers=(((0,), (0,)), ((), ())),
                preferred_element_type=f32)

        out_m = out_m + dotT(a_m, dot(x_m, w[0]))
        out_d = out_d + dotT(a_d, dot(x_d, w[1]))
        hm = jnp.concatenate(
            [dot(x_m, w[2 + k]) for k in range(3)], axis=1)
        rfull = dotT(t2, hm)
        row_k = jax.lax.broadcasted_iota(jnp.int32, (3 * _N_D, 3 * _DIM), 0) % 3
        col_k = jax.lax.broadcasted_iota(jnp.int32, (3 * _N_D, 3 * _DIM), 1) // _DIM
        sel = jnp.where(row_k == col_k, rfull, 0.0)
        folded = (sel[:, 0:_DIM] + sel[:, _DIM:2 * _DIM]
                  + sel[:, 2 * _DIM:3 * _DIM])
        out_d = out_d + folded.reshape(_N_D, 3, _DIM).sum(axis=1)
        h = jnp.stack([dot(x_d, w[5 + k]) for k in range(3)], axis=1)
        out_m = out_m + dot(t2, h.reshape(3 * _N_D, _DIM))

        b = bias_ref[...]
        x_m = jnp.maximum(out_m + b, 0.0)
        x_d = jnp.maximum(out_d + b, 0.0)

    outm_ref[...] = x_m
    outd_ref[...] = x_d


def kernel(m_sim, d_sim, ternary_association, node_embeddings,
           basis_w_0, coeff_0, self_loop_w_0, bias_0,
           basis_w_1, coeff_1, self_loop_w_1, bias_1):
    tern2d = ternary_association.reshape(_N_M, 3 * _N_D)
    x_m = node_embeddings[:_N_M]
    x_d = node_embeddings[_N_M:]
    bias0 = bias_0.reshape(1, _DIM)
    bias1 = bias_1.reshape(1, _DIM)

    vmem = pl.BlockSpec(memory_space=pltpu.VMEM)
    smem = pl.BlockSpec(memory_space=pltpu.SMEM)

    fn = pl.pallas_call(
        _rgcn_kernel,
        out_shape=(jax.ShapeDtypeStruct((_N_M, _DIM), jnp.float32),
                   jax.ShapeDtypeStruct((_N_D, _DIM), jnp.float32)),
        in_specs=[vmem] * 11 + [smem] * 2,
        out_specs=(vmem, vmem),
        compiler_params=pltpu.CompilerParams(
            vmem_limit_bytes=100 * 1024 * 1024),
    )
    return fn(m_sim, d_sim, tern2d, x_m, x_d,
              basis_w_0, self_loop_w_0, bias0,
              basis_w_1, self_loop_w_1, bias1,
              coeff_0, coeff_1)


# speedup vs baseline: 1.4410x; 1.4410x over previous
import jax
import jax.numpy as jnp
from jax.experimental import pallas as pl
from jax.experimental.pallas import tpu as pltpu

_N_M = 800
_N_D = 400
_DIM = 32
_N_REL = 8
_N_BASES = 4


def _rgcn_kernel(msim_ref, dsim_ref, t0_ref, t1_ref, t2_ref,
                 xm_ref, xd_ref,
                 basis0_ref, slw0_ref, bias0_ref,
                 basis1_ref, slw1_ref, bias1_ref,
                 coeff0_ref, coeff1_ref,
                 outm_ref, outd_ref):
    f32 = jnp.float32

    a_m = (msim_ref[...] > 0.3).astype(f32)
    a_d = (dsim_ref[...] > 0.3).astype(f32)
    t_fwd = [(t_ref[...] > 0.5).astype(f32)
             for t_ref in (t0_ref, t1_ref, t2_ref)]

    x_m = xm_ref[...]
    x_d = xd_ref[...]

    def dot(a, b):
        return jax.lax.dot(a, b, preferred_element_type=f32)

    for basis_ref, slw_ref, bias_ref, coeff_ref in (
            (basis0_ref, slw0_ref, bias0_ref, coeff0_ref),
            (basis1_ref, slw1_ref, bias1_ref, coeff1_ref)):
        w = []
        for r in range(_N_REL):
            wr = coeff_ref[r, 0] * basis_ref[0]
            for b in range(1, _N_BASES):
                wr = wr + coeff_ref[r, b] * basis_ref[b]
            w.append(wr)

        slw = slw_ref[...]
        out_m = dot(x_m, slw)
        out_d = dot(x_d, slw)
        def dotT(a, b):
            return jax.lax.dot_general(
                a, b, dimension_numbers=(((0,), (0,)), ((), ())),
                preferred_element_type=f32)

        out_m = out_m + dotT(a_m, dot(x_m, w[0]))
        out_d = out_d + dotT(a_d, dot(x_d, w[1]))
        for k in range(3):
            out_d = out_d + dotT(t_fwd[k], dot(x_m, w[2 + k]))
            out_m = out_m + dot(t_fwd[k], dot(x_d, w[5 + k]))

        b = bias_ref[...]
        x_m = jnp.maximum(out_m + b, 0.0)
        x_d = jnp.maximum(out_d + b, 0.0)

    outm_ref[...] = x_m
    outd_ref[...] = x_d


def kernel(m_sim, d_sim, ternary_association, node_embeddings,
           basis_w_0, coeff_0, self_loop_w_0, bias_0,
           basis_w_1, coeff_1, self_loop_w_1, bias_1):
    t0 = ternary_association[:, :, 0]
    t1 = ternary_association[:, :, 1]
    t2 = ternary_association[:, :, 2]
    x_m = node_embeddings[:_N_M]
    x_d = node_embeddings[_N_M:]
    bias0 = bias_0.reshape(1, _DIM)
    bias1 = bias_1.reshape(1, _DIM)

    vmem = pl.BlockSpec(memory_space=pltpu.VMEM)
    smem = pl.BlockSpec(memory_space=pltpu.SMEM)

    fn = pl.pallas_call(
        _rgcn_kernel,
        out_shape=(jax.ShapeDtypeStruct((_N_M, _DIM), jnp.float32),
                   jax.ShapeDtypeStruct((_N_D, _DIM), jnp.float32)),
        in_specs=[vmem] * 13 + [smem] * 2,
        out_specs=(vmem, vmem),
        compiler_params=pltpu.CompilerParams(
            vmem_limit_bytes=100 * 1024 * 1024),
    )
    return fn(m_sim, d_sim, t0, t1, t2, x_m, x_d,
              basis_w_0, self_loop_w_0, bias0,
              basis_w_1, self_loop_w_1, bias1,
              coeff_0, coeff_1)
